# baseline (speedup 1.0000x reference)
import functools

import jax
import jax.numpy as jnp
from jax import lax
from jax.experimental import pallas as pl
from jax.experimental.pallas import tpu as pltpu
from jax.experimental.pallas import tpu_sc as plsc

_N_NODES = 100000
_N_EDGES = 6400000
_PAD_NODES = 102400
_LANES = 16

_NW = 32
_C = 5120
_CHUNKS = _N_EDGES // _C
_MAX_PAIRS = (_CHUNKS // _NW + 2) // 2
_TBL = 100096


def _sigmoid_table(attn_log_logits):
    x = jnp.pad(attn_log_logits.reshape(-1), (0, _PAD_NODES - _N_NODES))
    x = x.reshape(800, 128)

    def body(x_ref, o_ref):
        o_ref[...] = jax.nn.sigmoid(x_ref[...])

    out = pl.pallas_call(
        body,
        out_shape=jax.ShapeDtypeStruct((800, 128), jnp.float32),
    )(x)
    return out.reshape(-1)


def _edge_attn_sc(table, edge_index):
    mesh = plsc.VectorSubcoreMesh(core_axis_name="c", subcore_axis_name="s")

    @functools.partial(
        pl.kernel,
        mesh=mesh,
        compiler_params=pltpu.CompilerParams(needs_layout_passes=False),
        out_type=jax.ShapeDtypeStruct((_N_EDGES,), jnp.float32),
        scratch_types=[
            pltpu.VMEM((_TBL,), jnp.float32),
            pltpu.VMEM((2, _C), jnp.int32),
            pltpu.VMEM((2, _C), jnp.int32),
            pltpu.VMEM((_C,), jnp.float32),
            pltpu.VMEM((_C,), jnp.float32),
            pltpu.SemaphoreType.DMA,
            pltpu.SemaphoreType.DMA,
            pltpu.SemaphoreType.DMA,
            pltpu.SemaphoreType.DMA,
        ],
    )
    def k(table_hbm, edges_hbm, out_hbm, table_v,
          e0, e1, o0, o1, smi0, smi1, smo0, smo1):
        wid = lax.axis_index("s") * 2 + lax.axis_index("c")
        e, o = (e0, e1), (o0, o1)
        smi, smo = (smi0, smi1), (smo0, smo1)

        def start_in(c, b):
            pltpu.async_copy(
                edges_hbm.at[:, pl.ds(c * _C, _C)], e[b], smi[b])

        def wait_in(b):
            pltpu.make_async_copy(
                edges_hbm.at[:, pl.ds(0, _C)], e[b], smi[b]).wait()

        def wait_out(b):
            pltpu.make_async_copy(
                o[b], out_hbm.at[pl.ds(0, _C)], smo[b]).wait()

        start_in(wid, 0)
        start_in(wid + _NW, 1)
        pltpu.sync_copy(table_hbm.at[pl.ds(0, _TBL)], table_v)

        def pair_body(p, carry):
            for b in range(2):
                j = p * 2 + b
                c = wid + j * _NW

                @pl.when(c < _CHUNKS)
                def _():
                    wait_in(b)

                    @pl.when(j >= 2)
                    def _():
                        wait_out(b)

                    @plsc.parallel_loop(0, _C, step=_LANES, unroll=32)
                    def _(off):
                        s = e[b][0, pl.ds(off, _LANES)]
                        d = e[b][1, pl.ds(off, _LANES)]
                        sv = plsc.load_gather(table_v, [s])
                        dv = plsc.load_gather(table_v, [d])
                        o[b][pl.ds(off, _LANES)] = sv * dv

                    pltpu.async_copy(
                        o[b], out_hbm.at[pl.ds(c * _C, _C)], smo[b])

                    @pl.when(c + 2 * _NW < _CHUNKS)
                    def _():
                        start_in(c + 2 * _NW, b)
            return carry

        lax.fori_loop(0, _MAX_PAIRS, pair_body, 0)
        wait_out(0)
        wait_out(1)

    return k(table, edge_index)


def kernel(attn_log_logits, edge_index):
    table = _sigmoid_table(attn_log_logits)
    out = _edge_attn_sc(table, edge_index)
    return out.reshape(_N_EDGES, 1)


# speedup vs baseline: 1.0045x; 1.0045x over previous
import functools

import jax
import jax.numpy as jnp
from jax import lax
from jax.experimental import pallas as pl
from jax.experimental.pallas import tpu as pltpu
from jax.experimental.pallas import tpu_sc as plsc

_N_NODES = 100000
_N_EDGES = 6400000
_PAD_NODES = 102400
_LANES = 16

_NW = 32
_C = 5120
_CHUNKS = _N_EDGES // _C
_MAX_PAIRS = (_CHUNKS // _NW + 2) // 2
_TBL = 100096


def _sigmoid_table(attn_log_logits):
    x = jnp.pad(attn_log_logits.reshape(-1), (0, _PAD_NODES - _N_NODES))
    x = x.reshape(800, 128)

    def body(x_ref, o_ref):
        o_ref[...] = jax.nn.sigmoid(x_ref[...])

    out = pl.pallas_call(
        body,
        out_shape=jax.ShapeDtypeStruct((800, 128), jnp.float32),
    )(x)
    return out.reshape(-1)


def _edge_attn_sc(table, edge_index):
    mesh = plsc.VectorSubcoreMesh(core_axis_name="c", subcore_axis_name="s")

    @functools.partial(
        pl.kernel,
        mesh=mesh,
        compiler_params=pltpu.CompilerParams(needs_layout_passes=False),
        out_type=jax.ShapeDtypeStruct((_N_EDGES,), jnp.float32),
        scratch_types=[
            pltpu.VMEM((_TBL,), jnp.float32),
            pltpu.VMEM((2, _C), jnp.int32),
            pltpu.VMEM((2, _C), jnp.int32),
            pltpu.VMEM((_C,), jnp.float32),
            pltpu.VMEM((_C,), jnp.float32),
            pltpu.SemaphoreType.DMA,
            pltpu.SemaphoreType.DMA,
            pltpu.SemaphoreType.DMA,
            pltpu.SemaphoreType.DMA,
        ],
    )
    def k(table_hbm, edges_hbm, out_hbm, table_v,
          e0, e1, o0, o1, smi0, smi1, smo0, smo1):
        wid = lax.axis_index("s") * 2 + lax.axis_index("c")
        e, o = (e0, e1), (o0, o1)
        smi, smo = (smi0, smi1), (smo0, smo1)

        def start_in(c, b):
            pltpu.async_copy(
                edges_hbm.at[:, pl.ds(c * _C, _C)], e[b], smi[b])

        def wait_in(b):
            pltpu.make_async_copy(
                edges_hbm.at[:, pl.ds(0, _C)], e[b], smi[b]).wait()

        def wait_out(b):
            pltpu.make_async_copy(
                o[b], out_hbm.at[pl.ds(0, _C)], smo[b]).wait()

        start_in(wid, 0)
        start_in(wid + _NW, 1)
        pltpu.sync_copy(table_hbm.at[pl.ds(0, _TBL)], table_v)

        def pair_body(p, carry):
            for b in range(2):
                j = p * 2 + b
                c = wid + j * _NW

                @pl.when(c < _CHUNKS)
                def _():
                    wait_in(b)

                    @pl.when(j >= 2)
                    def _():
                        wait_out(b)

                    @plsc.parallel_loop(0, _C, step=_LANES, unroll=16)
                    def _(off):
                        s = e[b][0, pl.ds(off, _LANES)]
                        d = e[b][1, pl.ds(off, _LANES)]
                        sv = plsc.load_gather(table_v, [s])
                        dv = plsc.load_gather(table_v, [d])
                        o[b][pl.ds(off, _LANES)] = sv * dv

                    pltpu.async_copy(
                        o[b], out_hbm.at[pl.ds(c * _C, _C)], smo[b])

                    @pl.when(c + 2 * _NW < _CHUNKS)
                    def _():
                        start_in(c + 2 * _NW, b)
            return carry

        lax.fori_loop(0, _MAX_PAIRS, pair_body, 0)
        wait_out(0)
        wait_out(1)

    return k(table, edge_index)


def kernel(attn_log_logits, edge_index):
    table = _sigmoid_table(attn_log_logits)
    out = _edge_attn_sc(table, edge_index)
    return out.reshape(_N_EDGES, 1)
